# baseline (speedup 1.0000x reference)
import functools

import jax
import jax.numpy as jnp
from jax import lax
from jax.experimental import pallas as pl
from jax.experimental.pallas import tpu as pltpu
from jax.experimental.pallas import tpu_sc as plsc

NQ = 1024
NS = 100000
DF = 512
K = 32
SB = 2048
NB = 49
NSP = NB * SB
CH = 128
NCH = NSP // CH
NCAND = K * CH
QB = 128
BIG = 1e30
IBIG = 1 << 30


def _dist_body(q_ref, s_ref, q2_ref, s2_ref, d2_ref, cmin_ref):
    j = pl.program_id(0)
    q = q_ref[...]
    s = s_ref[...]
    prod = lax.dot_general(q, s, (((1,), (1,)), ((), ())),
                           preferred_element_type=jnp.float32)
    d2 = q2_ref[...] + s2_ref[...] - 2.0 * prod
    col = j * SB + lax.broadcasted_iota(jnp.int32, (NQ, SB), 1)
    d2 = jnp.where(col >= NS, BIG, d2)
    for c in range(SB // CH):
        d2_ref[:, c * 8:(c + 1) * 8, :] = (
            d2[:, c * CH:(c + 1) * CH].reshape(NQ // 8, 8, CH))
    cmin_ref[0] = jnp.min(d2.reshape(NQ, SB // CH, CH), axis=2)


def _chunk_topk_body(cmin_ref, ids_ref):
    x = cmin_ref[...]
    col = lax.broadcasted_iota(jnp.int32, x.shape, 1)
    sels = []
    for _ in range(K):
        m = jnp.min(x, axis=1, keepdims=True)
        sel = jnp.min(jnp.where(x == m, col, IBIG), axis=1, keepdims=True)
        sels.append(sel)
        x = jnp.where(col == sel, jnp.float32(jnp.inf), x)
    ids = jnp.concatenate(sels, axis=1)
    row = lax.broadcasted_iota(jnp.int32, (NQ, K), 0)
    ids_ref[...] = (row // 8) * (NCH * 8) + ids * 8 + row % 8


def _final_body(g_ref, ids_ref, idx_ref, w_ref):
    qb = pl.program_id(0)
    vals = g_ref[...]
    flat = ids_ref[...]
    chunk = (flat % (NCH * 8)) // 8
    lane = lax.broadcasted_iota(jnp.int32, (QB, K, CH), 2)
    cand = (chunk[:, :, None] * CH + lane).reshape(QB, NCAND)
    dist = jnp.sqrt(jnp.maximum(vals, 0.0))
    idxs, dists = [], []
    for _ in range(K):
        m = jnp.min(dist, axis=1, keepdims=True)
        sel = jnp.min(jnp.where(dist == m, cand, IBIG), axis=1, keepdims=True)
        idxs.append(sel)
        dists.append(m)
        dist = jnp.where(cand == sel, jnp.float32(jnp.inf), dist)
    top_i = jnp.concatenate(idxs, axis=1)
    top_d = jnp.concatenate(dists, axis=1)
    sim = 1.0 / (top_d + 1e-6)
    w_ref[...] = sim / jnp.sum(sim, axis=1, keepdims=True)
    idx_ref[...] = top_i


def _make_sc_gather():
    info = plsc.get_sparse_core_info()
    nc, ns = info.num_cores, info.num_subcores
    nw = nc * ns
    total = NQ * K
    per_w = total // nw
    step = 128
    mesh = plsc.VectorSubcoreMesh(core_axis_name="c", subcore_axis_name="s")

    @functools.partial(
        pl.kernel, mesh=mesh,
        out_type=jax.ShapeDtypeStruct((total, CH), jnp.float32),
        scratch_types=[
            pltpu.VMEM((step,), jnp.int32),
            pltpu.VMEM((step, CH), jnp.float32),
            pltpu.SemaphoreType.DMA,
        ],
    )
    def gather(table_hbm, idx_hbm, out_hbm, idx_v, rows_v, sem):
        wid = lax.axis_index("s") * nc + lax.axis_index("c")
        base = wid * per_w
        for g in range(per_w // step):
            off = base + g * step
            pltpu.sync_copy(idx_hbm.at[pl.ds(off, step)], idx_v)
            pltpu.async_copy(table_hbm.at[idx_v], rows_v, sem).wait()
            pltpu.sync_copy(rows_v, out_hbm.at[pl.ds(off, step)])

    return gather


_sc_gather = None


def kernel(query, support):
    q = query.reshape(query.shape[0], -1)
    s = support.reshape(support.shape[0], -1)
    assert q.shape == (NQ, DF) and s.shape == (NS, DF)
    q2 = jnp.sum(q * q, axis=1, keepdims=True)
    s2 = jnp.sum(s * s, axis=1)[None, :]

    d2, cmin = pl.pallas_call(
        _dist_body,
        grid=(NB,),
        in_specs=[
            pl.BlockSpec((NQ, DF), lambda j: (0, 0)),
            pl.BlockSpec((SB, DF), lambda j: (j, 0)),
            pl.BlockSpec((NQ, 1), lambda j: (0, 0)),
            pl.BlockSpec((1, SB), lambda j: (0, j)),
        ],
        out_specs=[
            pl.BlockSpec((NQ // 8, SB // CH * 8, CH), lambda j: (0, j, 0)),
            pl.BlockSpec((1, NQ, SB // CH), lambda j: (j, 0, 0)),
        ],
        out_shape=[
            jax.ShapeDtypeStruct((NQ // 8, NCH * 8, CH), jnp.float32),
            jax.ShapeDtypeStruct((NB, NQ, SB // CH), jnp.float32),
        ],
        compiler_params=pltpu.CompilerParams(
            dimension_semantics=("arbitrary",),
        ),
    )(q, s, q2, s2)

    cmin = jnp.transpose(cmin, (1, 0, 2)).reshape(NQ, NCH)

    flat_ids = pl.pallas_call(
        _chunk_topk_body,
        in_specs=[pl.BlockSpec((NQ, NCH), lambda: (0, 0))],
        out_specs=pl.BlockSpec((NQ, K), lambda: (0, 0)),
        out_shape=jax.ShapeDtypeStruct((NQ, K), jnp.int32),
    )(cmin)

    global _sc_gather
    if _sc_gather is None:
        _sc_gather = _make_sc_gather()
    table = d2.reshape(NQ * NCH, CH)
    gathered = _sc_gather(table, flat_ids.reshape(NQ * K))

    indices, weights = pl.pallas_call(
        _final_body,
        grid=(NQ // QB,),
        in_specs=[
            pl.BlockSpec((QB, NCAND), lambda i: (i, 0)),
            pl.BlockSpec((QB, K), lambda i: (i, 0)),
        ],
        out_specs=[
            pl.BlockSpec((QB, K), lambda i: (i, 0)),
            pl.BlockSpec((QB, K), lambda i: (i, 0)),
        ],
        out_shape=[
            jax.ShapeDtypeStruct((NQ, K), jnp.int32),
            jax.ShapeDtypeStruct((NQ, K), jnp.float32),
        ],
        compiler_params=pltpu.CompilerParams(
            dimension_semantics=("arbitrary",),
        ),
    )(gathered.reshape(NQ, NCAND), flat_ids)

    return indices, weights


# speedup vs baseline: 1.5294x; 1.5294x over previous
import functools

import jax
import jax.numpy as jnp
from jax import lax
from jax.experimental import pallas as pl
from jax.experimental.pallas import tpu as pltpu
from jax.experimental.pallas import tpu_sc as plsc

NQ = 1024
NS = 100000
DF = 512
K = 32
SB = 2048
NB = 49
NSP = NB * SB
CH = 128
NCH = NSP // CH
NCAND = K * CH
QB = 128
BIG = 1e30
IBIG = 1 << 30


def _dist_body(q_ref, s_ref, q2_ref, d2_ref, cmin_ref):
    j = pl.program_id(0)
    q = q_ref[...]
    s = s_ref[...]
    s2 = jnp.sum(s * s, axis=1)
    prod = lax.dot_general(q, s, (((1,), (1,)), ((), ())),
                           preferred_element_type=jnp.float32)
    d2 = q2_ref[...] + s2[None, :] - 2.0 * prod
    col = j * SB + lax.broadcasted_iota(jnp.int32, (NQ, SB), 1)
    d2 = jnp.where(col >= NS, BIG, d2)
    for c in range(SB // CH):
        d2_ref[:, c * 8:(c + 1) * 8, :] = (
            d2[:, c * CH:(c + 1) * CH].reshape(NQ // 8, 8, CH))
    cmin_ref[0] = jnp.min(d2.reshape(NQ, SB // CH, CH), axis=2)


def _chunk_topk_body(cmin_ref, ids_ref):
    x = cmin_ref[...]
    col = lax.broadcasted_iota(jnp.int32, x.shape, 1)
    sels = []
    for _ in range(K):
        m = jnp.min(x, axis=1, keepdims=True)
        sel = jnp.min(jnp.where(x == m, col, IBIG), axis=1, keepdims=True)
        sels.append(sel)
        x = jnp.where(col == sel, jnp.float32(jnp.inf), x)
    ids = jnp.concatenate(sels, axis=1)
    row = lax.broadcasted_iota(jnp.int32, (NQ, K), 0)
    ids_ref[...] = (row // 8) * (NCH * 8) + ids * 8 + row % 8


def _final_body(g_ref, ids_ref, idx_ref, w_ref):
    qb = pl.program_id(0)
    vals = g_ref[...]
    flat = ids_ref[...]
    chunk = (flat % (NCH * 8)) // 8
    lane = lax.broadcasted_iota(jnp.int32, (QB, K, CH), 2)
    cand = (chunk[:, :, None] * CH + lane).reshape(QB, NCAND)
    dist = jnp.sqrt(jnp.maximum(vals, 0.0))
    idxs, dists = [], []
    for _ in range(K):
        m = jnp.min(dist, axis=1, keepdims=True)
        sel = jnp.min(jnp.where(dist == m, cand, IBIG), axis=1, keepdims=True)
        idxs.append(sel)
        dists.append(m)
        dist = jnp.where(cand == sel, jnp.float32(jnp.inf), dist)
    top_i = jnp.concatenate(idxs, axis=1)
    top_d = jnp.concatenate(dists, axis=1)
    sim = 1.0 / (top_d + 1e-6)
    w_ref[...] = sim / jnp.sum(sim, axis=1, keepdims=True)
    idx_ref[...] = top_i


def _make_sc_gather():
    info = plsc.get_sparse_core_info()
    nc, ns = info.num_cores, info.num_subcores
    nw = nc * ns
    total = NQ * K
    per_w = total // nw
    step = 128
    mesh = plsc.VectorSubcoreMesh(core_axis_name="c", subcore_axis_name="s")

    @functools.partial(
        pl.kernel, mesh=mesh,
        out_type=jax.ShapeDtypeStruct((total, CH), jnp.float32),
        scratch_types=[
            pltpu.VMEM((step,), jnp.int32),
            pltpu.VMEM((step, CH), jnp.float32),
            pltpu.SemaphoreType.DMA,
        ],
    )
    def gather(table_hbm, idx_hbm, out_hbm, idx_v, rows_v, sem):
        wid = lax.axis_index("s") * nc + lax.axis_index("c")
        base = wid * per_w
        for g in range(per_w // step):
            off = base + g * step
            pltpu.sync_copy(idx_hbm.at[pl.ds(off, step)], idx_v)
            pltpu.async_copy(table_hbm.at[idx_v], rows_v, sem).wait()
            pltpu.sync_copy(rows_v, out_hbm.at[pl.ds(off, step)])

    return gather


_sc_gather = None


def kernel(query, support):
    q = query.reshape(query.shape[0], -1)
    s = support.reshape(support.shape[0], -1)
    assert q.shape == (NQ, DF) and s.shape == (NS, DF)
    q2 = jnp.sum(q * q, axis=1, keepdims=True)

    d2, cmin = pl.pallas_call(
        _dist_body,
        grid=(NB,),
        in_specs=[
            pl.BlockSpec((NQ, DF), lambda j: (0, 0)),
            pl.BlockSpec((SB, DF), lambda j: (j, 0)),
            pl.BlockSpec((NQ, 1), lambda j: (0, 0)),
        ],
        out_specs=[
            pl.BlockSpec((NQ // 8, SB // CH * 8, CH), lambda j: (0, j, 0)),
            pl.BlockSpec((1, NQ, SB // CH), lambda j: (j, 0, 0)),
        ],
        out_shape=[
            jax.ShapeDtypeStruct((NQ // 8, NCH * 8, CH), jnp.float32),
            jax.ShapeDtypeStruct((NB, NQ, SB // CH), jnp.float32),
        ],
        compiler_params=pltpu.CompilerParams(
            dimension_semantics=("arbitrary",),
        ),
    )(q, s, q2)

    cmin = jnp.transpose(cmin, (1, 0, 2)).reshape(NQ, NCH)

    flat_ids = pl.pallas_call(
        _chunk_topk_body,
        in_specs=[pl.BlockSpec((NQ, NCH), lambda: (0, 0))],
        out_specs=pl.BlockSpec((NQ, K), lambda: (0, 0)),
        out_shape=jax.ShapeDtypeStruct((NQ, K), jnp.int32),
    )(cmin)

    global _sc_gather
    if _sc_gather is None:
        _sc_gather = _make_sc_gather()
    table = d2.reshape(NQ * NCH, CH)
    gathered = _sc_gather(table, flat_ids.reshape(NQ * K))

    indices, weights = pl.pallas_call(
        _final_body,
        grid=(NQ // QB,),
        in_specs=[
            pl.BlockSpec((QB, NCAND), lambda i: (i, 0)),
            pl.BlockSpec((QB, K), lambda i: (i, 0)),
        ],
        out_specs=[
            pl.BlockSpec((QB, K), lambda i: (i, 0)),
            pl.BlockSpec((QB, K), lambda i: (i, 0)),
        ],
        out_shape=[
            jax.ShapeDtypeStruct((NQ, K), jnp.int32),
            jax.ShapeDtypeStruct((NQ, K), jnp.float32),
        ],
        compiler_params=pltpu.CompilerParams(
            dimension_semantics=("arbitrary",),
        ),
    )(gathered.reshape(NQ, NCAND), flat_ids)

    return indices, weights
